# baseline (speedup 1.0000x reference)
import jax
import jax.numpy as jnp
from jax.experimental import pallas as pl
from jax.experimental.pallas import tpu as pltpu

_B = 8
_GRID = 16
_F32 = jnp.float32


def _dot(a, b):
    return jax.lax.dot_general(a, b, (((1,), (0,)), ((), ())),
                               preferred_element_type=_F32)


def _sel(rows, cols, stride, offset):
    r = jax.lax.broadcasted_iota(jnp.int32, (rows, cols), 0)
    c = jax.lax.broadcasted_iota(jnp.int32, (rows, cols), 1)
    return (r == stride * c + offset).astype(_F32)


def _cram3(x):
    c, b, t = x.shape
    z = jnp.zeros((c, b, 1), x.dtype)
    xp = jnp.concatenate([z, x, z], axis=2)
    cols = [xp[:, :, d:d + t] for d in range(3)]
    return jnp.concatenate(cols, axis=0).reshape(3 * c, b * t)


def _res_block(x, wa, wb):
    c, b, t = x.shape
    h = jax.nn.relu(x)
    h = jax.nn.relu(_dot(wa, _cram3(h)))
    h = _dot(wb, h)
    return x + h.reshape(c, b, t)


def _fwd(xc_ref, w1_ref, b1_ref, w2_ref, b2_ref, w3_ref, b3_ref,
         er1a_ref, er1b_ref, er2a_ref, er2b_ref, wp_ref, bp_ref, e_ref,
         dc1_ref, dc1b_ref, dr1a_ref, dr1b_ref, dr2a_ref, dr2b_ref,
         vt_ref, t1b_ref, ut_ref, t2b_ref, wadj_ref, badj_ref,
         loss_ref, xout_ref, perp_ref, acc_sq, acc_hist):
    pid = pl.program_id(0)
    B = _B

    xc = xc_ref[...].reshape(240, B * 128)
    h = jax.nn.relu(_dot(w1_ref[...], xc) + b1_ref[...])
    h = h.reshape(64, B, 128)

    zp = jnp.zeros((64, B, 1), _F32)
    xp = jnp.concatenate([zp, h, zp], axis=2)
    xf = xp.reshape(64 * B, 130)
    phases = [jnp.dot(xf, _sel(130, 64, 2, d),
                      preferred_element_type=_F32).reshape(64, B, 64)
              for d in range(4)]
    xi = jnp.concatenate(phases, axis=0).reshape(256, B * 64)
    h = jax.nn.relu(_dot(w2_ref[...], xi) + b2_ref[...])
    h = h.reshape(128, B, 64)

    h = (_dot(w3_ref[...], _cram3(h)) + b3_ref[...]).reshape(128, B, 64)

    h = _res_block(h, er1a_ref[...], er1b_ref[...])
    h = _res_block(h, er2a_ref[...], er2b_ref[...])
    h = jax.nn.relu(h)

    z = _dot(wp_ref[...], h.reshape(128, B * 64)) + bp_ref[...]

    e = e_ref[...]
    esq = jnp.sum(e * e, axis=1, keepdims=True)
    zsq = jnp.sum(z * z, axis=0, keepdims=True)
    s = esq + zsq - 2.0 * _dot(e, z)
    idx = jnp.argmin(s, axis=0)
    iota = jax.lax.broadcasted_iota(jnp.int32, s.shape, 0)
    hot = (iota == idx[None, :]).astype(_F32)
    q = jax.lax.dot_general(e, hot, (((0,), (0,)), ((), ())),
                            preferred_element_type=_F32)

    sq = jnp.sum((q - z) ** 2)
    hist = jnp.sum(hot, axis=1, keepdims=True)

    @pl.when(pid == 0)
    def _init():
        acc_sq[...] = jnp.zeros((1, 1), _F32)
        acc_hist[...] = jnp.zeros((128, 1), _F32)

    acc_sq[...] += sq.reshape(1, 1)
    acc_hist[...] += hist

    h = q.reshape(128, B, 64)
    h = (_dot(dc1_ref[...], _cram3(h)) + dc1b_ref[...]).reshape(128, B, 64)

    h = _res_block(h, dr1a_ref[...], dr1b_ref[...])
    h = _res_block(h, dr2a_ref[...], dr2b_ref[...])
    h = jax.nn.relu(h)

    xf = h.reshape(128, B * 64)
    p0 = _dot(vt_ref[0], xf).reshape(64, B, 64)
    p1 = _dot(vt_ref[1], xf).reshape(64, B, 64)
    p2 = _dot(vt_ref[2], xf).reshape(64, B, 64)
    p3 = _dot(vt_ref[3], xf).reshape(64, B, 64)
    zc = jnp.zeros((64, B, 1), _F32)
    p3s = jnp.concatenate([zc, p3[:, :, :63]], axis=2)
    p0s = jnp.concatenate([p0[:, :, 1:], zc], axis=2)
    y_even = (p1 + p3s).reshape(64 * B, 64)
    y_odd = (p0s + p2).reshape(64 * B, 64)
    y = (jnp.dot(y_even, _sel(128, 64, 2, 0).T, preferred_element_type=_F32)
         + jnp.dot(y_odd, _sel(128, 64, 2, 1).T, preferred_element_type=_F32))
    h = jax.nn.relu(y.reshape(64, B * 128) + t1b_ref[...]).reshape(64, B, 128)

    xf = h.reshape(64, B * 128)
    y = jnp.zeros((60 * B, 512), _F32)
    for d in range(4):
        pd = _dot(ut_ref[d], xf).reshape(60 * B, 128)
        y = y + jnp.dot(pd, _sel(512, 128, 4, d).T,
                        preferred_element_type=_F32)
    h = (y.reshape(60, B * 512) + t2b_ref[...]).reshape(60, B, 512)

    z3 = jnp.zeros((60, B, 3), _F32)
    xp = jnp.concatenate([z3, h, z3], axis=2)
    cols = [xp[:, :, d:d + 516] for d in range(3)]
    xi = jnp.concatenate(cols, axis=0).reshape(180, B * 516)
    out = _dot(wadj_ref[...], xi) + badj_ref[...]
    xout_ref[...] = out.reshape(60, B, 516)

    @pl.when(pid == _GRID - 1)
    def _finalize():
        total = 128.0 * 64.0 * 128.0
        loss_ref[...] = 2.0 * acc_sq[...] / total
        probs = acc_hist[...] / 8192.0
        ent = jnp.sum(probs * jnp.log(probs + 1e-10))
        perp_ref[...] = jnp.exp(-ent).reshape(1, 1)


def _full(shape):
    nd = len(shape)
    return pl.BlockSpec(shape, lambda i: (0,) * nd)


def kernel(x, params):
    p = params
    xt = jnp.transpose(x, (2, 0, 1))
    xp = jnp.pad(xt, ((0, 0), (0, 0), (1, 0)))
    xcol = jnp.concatenate([xp[:, :, d::4][:, :, :128] for d in range(4)],
                           axis=0)

    w1 = p['enc_c1_w'].transpose(0, 2, 1).reshape(64, 240)
    w2 = p['enc_c2_w'].transpose(0, 2, 1).reshape(128, 256)
    w3 = p['enc_c3_w'].transpose(0, 2, 1).reshape(128, 384)
    er = p['enc_res']
    er1a = er[0][0].transpose(0, 2, 1).reshape(32, 384)
    er1b = er[0][1].reshape(128, 32)
    er2a = er[1][0].transpose(0, 2, 1).reshape(32, 384)
    er2b = er[1][1].reshape(128, 32)
    wp = p['pre_w'].reshape(128, 128)
    dc1 = p['dec_c1_w'].transpose(0, 2, 1).reshape(128, 384)
    dr = p['dec_res']
    dr1a = dr[0][0].transpose(0, 2, 1).reshape(32, 384)
    dr1b = dr[0][1].reshape(128, 32)
    dr2a = dr[1][0].transpose(0, 2, 1).reshape(32, 384)
    dr2b = dr[1][1].reshape(128, 32)
    vt = jnp.stack([p['dec_t1_w'][:, :, d].T for d in range(4)])
    ut = jnp.stack([p['dec_t2_w'][:, :, d].T for d in range(4)])
    wadj = p['dec_adj_w'].transpose(0, 2, 1).reshape(60, 180)

    b1 = p['enc_c1_b'].reshape(64, 1)
    b2 = p['enc_c2_b'].reshape(128, 1)
    b3 = p['enc_c3_b'].reshape(128, 1)
    bp = p['pre_b'].reshape(128, 1)
    dc1b = p['dec_c1_b'].reshape(128, 1)
    t1b = p['dec_t1_b'].reshape(64, 1)
    t2b = p['dec_t2_b'].reshape(60, 1)
    badj = p['dec_adj_b'].reshape(60, 1)

    ins = [xcol, w1, b1, w2, b2, w3, b3, er1a, er1b, er2a, er2b, wp, bp,
           p['codebook'], dc1, dc1b, dr1a, dr1b, dr2a, dr2b,
           vt, t1b, ut, t2b, wadj, badj]

    in_specs = [pl.BlockSpec((240, _B, 128), lambda i: (0, i, 0))]
    in_specs += [_full(a.shape) for a in ins[1:]]

    out_shape = [jax.ShapeDtypeStruct((1, 1), _F32),
                 jax.ShapeDtypeStruct((60, 128, 516), _F32),
                 jax.ShapeDtypeStruct((1, 1), _F32)]
    out_specs = [pl.BlockSpec((1, 1), lambda i: (0, 0)),
                 pl.BlockSpec((60, _B, 516), lambda i: (0, i, 0)),
                 pl.BlockSpec((1, 1), lambda i: (0, 0))]

    loss, xo, perp = pl.pallas_call(
        _fwd,
        grid=(_GRID,),
        in_specs=in_specs,
        out_specs=out_specs,
        out_shape=out_shape,
        scratch_shapes=[pltpu.VMEM((1, 1), _F32),
                        pltpu.VMEM((128, 1), _F32)],
    )(*ins)

    return loss[0, 0], jnp.transpose(xo, (1, 2, 0)), perp[0, 0]


# speedup vs baseline: 1.0446x; 1.0446x over previous
import jax
import jax.numpy as jnp
from jax.experimental import pallas as pl
from jax.experimental.pallas import tpu as pltpu

_B = 16
_GRID = 8
_F32 = jnp.float32


def _dot(a, b):
    return jax.lax.dot_general(a, b, (((1,), (0,)), ((), ())),
                               preferred_element_type=_F32)


def _sel(rows, cols, stride, offset):
    r = jax.lax.broadcasted_iota(jnp.int32, (rows, cols), 0)
    c = jax.lax.broadcasted_iota(jnp.int32, (rows, cols), 1)
    return (r == stride * c + offset).astype(_F32)


def _cram3(x):
    c, b, t = x.shape
    z = jnp.zeros((c, b, 1), x.dtype)
    xp = jnp.concatenate([z, x, z], axis=2)
    cols = [xp[:, :, d:d + t] for d in range(3)]
    return jnp.concatenate(cols, axis=0).reshape(3 * c, b * t)


def _res_block(x, wa, wb):
    c, b, t = x.shape
    h = jax.nn.relu(x)
    h = jax.nn.relu(_dot(wa, _cram3(h)))
    h = _dot(wb, h)
    return x + h.reshape(c, b, t)


def _fwd(xc_ref, w1_ref, b1_ref, w2_ref, b2_ref, w3_ref, b3_ref,
         er1a_ref, er1b_ref, er2a_ref, er2b_ref, wp_ref, bp_ref, e_ref,
         dc1_ref, dc1b_ref, dr1a_ref, dr1b_ref, dr2a_ref, dr2b_ref,
         vt_ref, t1b_ref, ut_ref, t2b_ref, wadj_ref, badj_ref,
         loss_ref, xout_ref, perp_ref, acc_sq, acc_hist):
    pid = pl.program_id(0)
    B = _B

    xc = xc_ref[...].reshape(240, B * 128)
    h = jax.nn.relu(_dot(w1_ref[...], xc) + b1_ref[...])
    h = h.reshape(64, B, 128)

    zp = jnp.zeros((64, B, 1), _F32)
    xp = jnp.concatenate([zp, h, zp], axis=2)
    xf = xp.reshape(64 * B, 130)
    phases = [jnp.dot(xf, _sel(130, 64, 2, d),
                      preferred_element_type=_F32).reshape(64, B, 64)
              for d in range(4)]
    xi = jnp.concatenate(phases, axis=0).reshape(256, B * 64)
    h = jax.nn.relu(_dot(w2_ref[...], xi) + b2_ref[...])
    h = h.reshape(128, B, 64)

    h = (_dot(w3_ref[...], _cram3(h)) + b3_ref[...]).reshape(128, B, 64)

    h = _res_block(h, er1a_ref[...], er1b_ref[...])
    h = _res_block(h, er2a_ref[...], er2b_ref[...])
    h = jax.nn.relu(h)

    z = _dot(wp_ref[...], h.reshape(128, B * 64)) + bp_ref[...]

    e = e_ref[...]
    esq = jnp.sum(e * e, axis=1, keepdims=True)
    zsq = jnp.sum(z * z, axis=0, keepdims=True)
    s = esq + zsq - 2.0 * _dot(e, z)
    idx = jnp.argmin(s, axis=0)
    iota = jax.lax.broadcasted_iota(jnp.int32, s.shape, 0)
    hot = (iota == idx[None, :]).astype(_F32)
    q = jax.lax.dot_general(e, hot, (((0,), (0,)), ((), ())),
                            preferred_element_type=_F32)

    sq = jnp.sum((q - z) ** 2)
    hist = jnp.sum(hot, axis=1, keepdims=True)

    @pl.when(pid == 0)
    def _init():
        acc_sq[...] = jnp.zeros((1, 1), _F32)
        acc_hist[...] = jnp.zeros((128, 1), _F32)

    acc_sq[...] += sq.reshape(1, 1)
    acc_hist[...] += hist

    h = q.reshape(128, B, 64)
    h = (_dot(dc1_ref[...], _cram3(h)) + dc1b_ref[...]).reshape(128, B, 64)

    h = _res_block(h, dr1a_ref[...], dr1b_ref[...])
    h = _res_block(h, dr2a_ref[...], dr2b_ref[...])
    h = jax.nn.relu(h)

    xf = h.reshape(128, B * 64)
    p0 = _dot(vt_ref[0], xf).reshape(64, B, 64)
    p1 = _dot(vt_ref[1], xf).reshape(64, B, 64)
    p2 = _dot(vt_ref[2], xf).reshape(64, B, 64)
    p3 = _dot(vt_ref[3], xf).reshape(64, B, 64)
    zc = jnp.zeros((64, B, 1), _F32)
    p3s = jnp.concatenate([zc, p3[:, :, :63]], axis=2)
    p0s = jnp.concatenate([p0[:, :, 1:], zc], axis=2)
    y_even = (p1 + p3s).reshape(64 * B, 64)
    y_odd = (p0s + p2).reshape(64 * B, 64)
    y = (jnp.dot(y_even, _sel(128, 64, 2, 0).T, preferred_element_type=_F32)
         + jnp.dot(y_odd, _sel(128, 64, 2, 1).T, preferred_element_type=_F32))
    h = jax.nn.relu(y.reshape(64, B * 128) + t1b_ref[...]).reshape(64, B, 128)

    xf = h.reshape(64, B * 128)
    y = jnp.zeros((60 * B, 512), _F32)
    for d in range(4):
        pd = _dot(ut_ref[d], xf).reshape(60 * B, 128)
        y = y + jnp.dot(pd, _sel(512, 128, 4, d).T,
                        preferred_element_type=_F32)
    h = (y.reshape(60, B * 512) + t2b_ref[...]).reshape(60, B, 512)

    z3 = jnp.zeros((60, B, 3), _F32)
    xp = jnp.concatenate([z3, h, z3], axis=2)
    cols = [xp[:, :, d:d + 516] for d in range(3)]
    xi = jnp.concatenate(cols, axis=0).reshape(180, B * 516)
    out = _dot(wadj_ref[...], xi) + badj_ref[...]
    xout_ref[...] = out.reshape(60, B, 516)

    @pl.when(pid == _GRID - 1)
    def _finalize():
        total = 128.0 * 64.0 * 128.0
        loss_ref[...] = 2.0 * acc_sq[...] / total
        probs = acc_hist[...] / 8192.0
        ent = jnp.sum(probs * jnp.log(probs + 1e-10))
        perp_ref[...] = jnp.exp(-ent).reshape(1, 1)


def _full(shape):
    nd = len(shape)
    return pl.BlockSpec(shape, lambda i: (0,) * nd)


def kernel(x, params):
    p = params
    xt = jnp.transpose(x, (2, 0, 1))
    xp = jnp.pad(xt, ((0, 0), (0, 0), (1, 0)))
    xcol = jnp.concatenate([xp[:, :, d::4][:, :, :128] for d in range(4)],
                           axis=0)

    w1 = p['enc_c1_w'].transpose(0, 2, 1).reshape(64, 240)
    w2 = p['enc_c2_w'].transpose(0, 2, 1).reshape(128, 256)
    w3 = p['enc_c3_w'].transpose(0, 2, 1).reshape(128, 384)
    er = p['enc_res']
    er1a = er[0][0].transpose(0, 2, 1).reshape(32, 384)
    er1b = er[0][1].reshape(128, 32)
    er2a = er[1][0].transpose(0, 2, 1).reshape(32, 384)
    er2b = er[1][1].reshape(128, 32)
    wp = p['pre_w'].reshape(128, 128)
    dc1 = p['dec_c1_w'].transpose(0, 2, 1).reshape(128, 384)
    dr = p['dec_res']
    dr1a = dr[0][0].transpose(0, 2, 1).reshape(32, 384)
    dr1b = dr[0][1].reshape(128, 32)
    dr2a = dr[1][0].transpose(0, 2, 1).reshape(32, 384)
    dr2b = dr[1][1].reshape(128, 32)
    vt = jnp.stack([p['dec_t1_w'][:, :, d].T for d in range(4)])
    ut = jnp.stack([p['dec_t2_w'][:, :, d].T for d in range(4)])
    wadj = p['dec_adj_w'].transpose(0, 2, 1).reshape(60, 180)

    b1 = p['enc_c1_b'].reshape(64, 1)
    b2 = p['enc_c2_b'].reshape(128, 1)
    b3 = p['enc_c3_b'].reshape(128, 1)
    bp = p['pre_b'].reshape(128, 1)
    dc1b = p['dec_c1_b'].reshape(128, 1)
    t1b = p['dec_t1_b'].reshape(64, 1)
    t2b = p['dec_t2_b'].reshape(60, 1)
    badj = p['dec_adj_b'].reshape(60, 1)

    ins = [xcol, w1, b1, w2, b2, w3, b3, er1a, er1b, er2a, er2b, wp, bp,
           p['codebook'], dc1, dc1b, dr1a, dr1b, dr2a, dr2b,
           vt, t1b, ut, t2b, wadj, badj]

    in_specs = [pl.BlockSpec((240, _B, 128), lambda i: (0, i, 0))]
    in_specs += [_full(a.shape) for a in ins[1:]]

    out_shape = [jax.ShapeDtypeStruct((1, 1), _F32),
                 jax.ShapeDtypeStruct((60, 128, 516), _F32),
                 jax.ShapeDtypeStruct((1, 1), _F32)]
    out_specs = [pl.BlockSpec((1, 1), lambda i: (0, 0)),
                 pl.BlockSpec((60, _B, 516), lambda i: (0, i, 0)),
                 pl.BlockSpec((1, 1), lambda i: (0, 0))]

    loss, xo, perp = pl.pallas_call(
        _fwd,
        grid=(_GRID,),
        in_specs=in_specs,
        out_specs=out_specs,
        out_shape=out_shape,
        scratch_shapes=[pltpu.VMEM((1, 1), _F32),
                        pltpu.VMEM((128, 1), _F32)],
    )(*ins)

    return loss[0, 0], jnp.transpose(xo, (1, 2, 0)), perp[0, 0]


# speedup vs baseline: 2.9246x; 2.7997x over previous
import jax
import jax.numpy as jnp
from jax.experimental import pallas as pl
from jax.experimental.pallas import tpu as pltpu

_B = 16
_GRID = 8
_F32 = jnp.float32


def _dot(a, b):
    return jax.lax.dot_general(a, b, (((1,), (0,)), ((), ())),
                               preferred_element_type=_F32)


def _sel(rows, cols, stride, offset):
    r = jax.lax.broadcasted_iota(jnp.int32, (rows, cols), 0)
    c = jax.lax.broadcasted_iota(jnp.int32, (rows, cols), 1)
    return (r == stride * c + offset).astype(_F32)


def _conv3(x, w3, bias=None):
    c, b, t = x.shape
    xf = x.reshape(c, b * t)
    z0 = _dot(w3[0], xf).reshape(-1, b, t)
    z1 = _dot(w3[1], xf).reshape(-1, b, t)
    z2 = _dot(w3[2], xf).reshape(-1, b, t)
    o = z0.shape[0]
    zc = jnp.zeros((o, b, 1), _F32)
    y = z1 + jnp.concatenate([zc, z0[:, :, :t - 1]], axis=2)
    y = y + jnp.concatenate([z2[:, :, 1:], zc], axis=2)
    if bias is not None:
        y = y + bias[:, None, :1]
    return y


def _res_block(x, wa3, wb):
    c, b, t = x.shape
    h = jax.nn.relu(x)
    h = jax.nn.relu(_conv3(h, wa3))
    h = _dot(wb, h.reshape(-1, b * t))
    return x + h.reshape(c, b, t)


def _fwd(xc_ref, w1t_ref, b1_ref, w2_ref, b2_ref, w33_ref, b3_ref,
         er1a_ref, er1b_ref, er2a_ref, er2b_ref, wp_ref, bp_ref, e_ref,
         dc13_ref, dc1b_ref, dr1a_ref, dr1b_ref, dr2a_ref, dr2b_ref,
         vt_ref, t1b_ref, ut_ref, t2b_ref, wadj3_ref, badj_ref,
         loss_ref, xout_ref, perp_ref, acc_sq, acc_hist):
    pid = pl.program_id(0)
    B = _B

    rows = xc_ref[...].reshape(B * 128, 240)
    y1 = jax.nn.relu(jnp.dot(rows, w1t_ref[...],
                             preferred_element_type=_F32) + b1_ref[...])
    h = jnp.transpose(y1).reshape(64, B, 128)

    zp = jnp.zeros((64, B, 1), _F32)
    xp = jnp.concatenate([zp, h, zp], axis=2)
    xf = xp.reshape(64 * B, 130)
    phases = [jnp.dot(xf, _sel(130, 64, 2, d),
                      preferred_element_type=_F32).reshape(64, B, 64)
              for d in range(4)]
    xi = jnp.concatenate(phases, axis=0).reshape(256, B * 64)
    h = jax.nn.relu(_dot(w2_ref[...], xi) + b2_ref[...])
    h = h.reshape(128, B, 64)

    h = _conv3(h, w33_ref[...], b3_ref[...])
    h = _res_block(h, er1a_ref[...], er1b_ref[...])
    h = _res_block(h, er2a_ref[...], er2b_ref[...])
    h = jax.nn.relu(h)

    z = _dot(wp_ref[...], h.reshape(128, B * 64)) + bp_ref[...]

    e = e_ref[...]
    esq = jnp.sum(e * e, axis=1, keepdims=True)
    zsq = jnp.sum(z * z, axis=0, keepdims=True)
    s = esq + zsq - 2.0 * _dot(e, z)
    idx = jnp.argmin(s, axis=0)
    iota = jax.lax.broadcasted_iota(jnp.int32, s.shape, 0)
    hot = (iota == idx[None, :]).astype(_F32)
    q = jax.lax.dot_general(e, hot, (((0,), (0,)), ((), ())),
                            preferred_element_type=_F32)

    sq = jnp.sum((q - z) ** 2)
    hist = jnp.sum(hot, axis=1, keepdims=True)

    @pl.when(pid == 0)
    def _init():
        acc_sq[...] = jnp.zeros((1, 1), _F32)
        acc_hist[...] = jnp.zeros((128, 1), _F32)

    acc_sq[...] += sq.reshape(1, 1)
    acc_hist[...] += hist

    h = q.reshape(128, B, 64)
    h = _conv3(h, dc13_ref[...], dc1b_ref[...])
    h = _res_block(h, dr1a_ref[...], dr1b_ref[...])
    h = _res_block(h, dr2a_ref[...], dr2b_ref[...])
    h = jax.nn.relu(h)

    xf = h.reshape(128, B * 64)
    p0 = _dot(vt_ref[0], xf).reshape(64, B, 64)
    p1 = _dot(vt_ref[1], xf).reshape(64, B, 64)
    p2 = _dot(vt_ref[2], xf).reshape(64, B, 64)
    p3 = _dot(vt_ref[3], xf).reshape(64, B, 64)
    zc = jnp.zeros((64, B, 1), _F32)
    p3s = jnp.concatenate([zc, p3[:, :, :63]], axis=2)
    p0s = jnp.concatenate([p0[:, :, 1:], zc], axis=2)
    y_even = (p1 + p3s).reshape(64 * B, 64)
    y_odd = (p0s + p2).reshape(64 * B, 64)
    y = (jnp.dot(y_even, _sel(128, 64, 2, 0).T, preferred_element_type=_F32)
         + jnp.dot(y_odd, _sel(128, 64, 2, 1).T, preferred_element_type=_F32))
    h = jax.nn.relu(y.reshape(64, B * 128) + t1b_ref[...]).reshape(64, B, 128)

    xf = h.reshape(64, B * 128)
    y = jnp.zeros((60 * B, 512), _F32)
    for d in range(4):
        pd = _dot(ut_ref[d], xf).reshape(60 * B, 128)
        y = y + jnp.dot(pd, _sel(512, 128, 4, d).T,
                        preferred_element_type=_F32)
    h = (y.reshape(60, B * 512) + t2b_ref[...]).reshape(60, B, 512)

    xf = h.reshape(60, B * 512)
    out = None
    for d in range(3):
        zd = _dot(wadj3_ref[d], xf).reshape(60, B, 512)
        zl = jnp.zeros((60, B, 3 - d), _F32)
        zr = jnp.zeros((60, B, 1 + d), _F32)
        term = jnp.concatenate([zl, zd, zr], axis=2)
        out = term if out is None else out + term
    out = out.reshape(60, B * 516) + badj_ref[...]
    xout_ref[...] = jnp.transpose(out).reshape(B, 516, 60)

    @pl.when(pid == _GRID - 1)
    def _finalize():
        total = 128.0 * 64.0 * 128.0
        loss_ref[...] = 2.0 * acc_sq[...] / total
        probs = acc_hist[...] / 8192.0
        ent = jnp.sum(probs * jnp.log(probs + 1e-10))
        perp_ref[...] = jnp.exp(-ent).reshape(1, 1)


def _full(shape):
    nd = len(shape)
    return pl.BlockSpec(shape, lambda i: (0,) * nd)


def _taps3(w):
    return jnp.transpose(w, (2, 0, 1))


def kernel(x, params):
    p = params
    n = x.shape[0]
    xp = jnp.concatenate([jnp.zeros((n, 1, 60), x.dtype), x[:, :511, :]],
                         axis=1)
    xcol = xp.reshape(n, 128, 240)

    w1t = jnp.transpose(p['enc_c1_w'], (2, 1, 0)).reshape(240, 64)
    w2 = p['enc_c2_w'].transpose(0, 2, 1).reshape(128, 256)
    w33 = _taps3(p['enc_c3_w'])
    er = p['enc_res']
    er1a = _taps3(er[0][0])
    er1b = er[0][1].reshape(128, 32)
    er2a = _taps3(er[1][0])
    er2b = er[1][1].reshape(128, 32)
    wp = p['pre_w'].reshape(128, 128)
    dc13 = _taps3(p['dec_c1_w'])
    dr = p['dec_res']
    dr1a = _taps3(dr[0][0])
    dr1b = dr[0][1].reshape(128, 32)
    dr2a = _taps3(dr[1][0])
    dr2b = dr[1][1].reshape(128, 32)
    vt = jnp.stack([p['dec_t1_w'][:, :, d].T for d in range(4)])
    ut = jnp.stack([p['dec_t2_w'][:, :, d].T for d in range(4)])
    wadj3 = _taps3(p['dec_adj_w'])

    b1 = p['enc_c1_b'].reshape(1, 64)
    b2 = p['enc_c2_b'].reshape(128, 1)
    b3 = p['enc_c3_b'].reshape(128, 1)
    bp = p['pre_b'].reshape(128, 1)
    dc1b = p['dec_c1_b'].reshape(128, 1)
    t1b = p['dec_t1_b'].reshape(64, 1)
    t2b = p['dec_t2_b'].reshape(60, 1)
    badj = p['dec_adj_b'].reshape(60, 1)

    ins = [xcol, w1t, b1, w2, b2, w33, b3, er1a, er1b, er2a, er2b, wp, bp,
           p['codebook'], dc13, dc1b, dr1a, dr1b, dr2a, dr2b,
           vt, t1b, ut, t2b, wadj3, badj]

    in_specs = [pl.BlockSpec((_B, 128, 240), lambda i: (i, 0, 0))]
    in_specs += [_full(a.shape) for a in ins[1:]]

    out_shape = [jax.ShapeDtypeStruct((1, 1), _F32),
                 jax.ShapeDtypeStruct((128, 516, 60), _F32),
                 jax.ShapeDtypeStruct((1, 1), _F32)]
    out_specs = [pl.BlockSpec((1, 1), lambda i: (0, 0)),
                 pl.BlockSpec((_B, 516, 60), lambda i: (i, 0, 0)),
                 pl.BlockSpec((1, 1), lambda i: (0, 0))]

    loss, xo, perp = pl.pallas_call(
        _fwd,
        grid=(_GRID,),
        in_specs=in_specs,
        out_specs=out_specs,
        out_shape=out_shape,
        scratch_shapes=[pltpu.VMEM((1, 1), _F32),
                        pltpu.VMEM((128, 1), _F32)],
    )(*ins)

    return loss[0, 0], xo, perp[0, 0]


# speedup vs baseline: 3.0888x; 1.0561x over previous
import jax
import jax.numpy as jnp
from jax.experimental import pallas as pl
from jax.experimental.pallas import tpu as pltpu

_B = 16
_GRID = 8
_F32 = jnp.float32


def _dot(a, b):
    return jax.lax.dot_general(a, b, (((1,), (0,)), ((), ())),
                               preferred_element_type=_F32)


def _dotT(a, b):
    return jax.lax.dot_general(a, b, (((0,), (0,)), ((), ())),
                               preferred_element_type=_F32)


def _iotas(shape):
    return (jax.lax.broadcasted_iota(jnp.int32, shape, 0),
            jax.lax.broadcasted_iota(jnp.int32, shape, 1))


def _sel(rows, cols, stride, offset):
    r, c = _iotas((rows, cols))
    return (r == stride * c + offset).astype(_F32)


def _taps3(flat):
    o, c3 = flat.shape
    c = c3 // 3
    r, cc = _iotas((c3, c))
    return [_dot(flat, ((r == 3 * cc + d)).astype(_F32)) for d in range(3)]


def _conv3(x, taps, bias=None):
    c, b, t = x.shape
    xf = x.reshape(c, b * t)
    z0 = _dot(taps[0], xf).reshape(-1, b, t)
    z1 = _dot(taps[1], xf).reshape(-1, b, t)
    z2 = _dot(taps[2], xf).reshape(-1, b, t)
    o = z0.shape[0]
    zc = jnp.zeros((o, b, 1), _F32)
    y = z1 + jnp.concatenate([zc, z0[:, :, :t - 1]], axis=2)
    y = y + jnp.concatenate([z2[:, :, 1:], zc], axis=2)
    if bias is not None:
        y = y + bias[:, None, :1]
    return y


def _res_block(x, wa_flat, wb):
    c, b, t = x.shape
    h = jax.nn.relu(x)
    h = jax.nn.relu(_conv3(h, _taps3(wa_flat)))
    h = _dot(wb, h.reshape(-1, b * t))
    return x + h.reshape(c, b, t)


def _fwd(xc_ref, w1_ref, b1_ref, w2_ref, b2_ref, w3_ref, b3_ref,
         er1a_ref, er1b_ref, er2a_ref, er2b_ref, wp_ref, bp_ref, e_ref,
         dc1_ref, dc1b_ref, dr1a_ref, dr1b_ref, dr2a_ref, dr2b_ref,
         wt1_ref, t1b_ref, wt2_ref, t2b_ref, wadj_ref, badj_ref,
         loss_ref, xout_ref, perp_ref, acc_sq, acc_hist):
    pid = pl.program_id(0)
    B = _B

    w1t = jnp.transpose(w1_ref[...])
    r, c = _iotas((240, 240))
    sel_a = ((c == (r % 60) * 4 + r // 60 + 1) & (r // 60 < 3)).astype(_F32)
    sel_b = ((c == (r % 60) * 4) & (r // 60 == 3)).astype(_F32)
    a_mat = _dot(sel_a, w1t)
    b_mat = _dot(sel_b, w1t)
    rows = xc_ref[...].reshape(B * 128, 240)
    rsh = jnp.concatenate([jnp.zeros((1, 240), _F32), rows[:B * 128 - 1, :]],
                          axis=0)
    rmask = (jax.lax.broadcasted_iota(jnp.int32, (B * 128, 1), 0) % 128
             != 0).astype(_F32)
    y1 = (jnp.dot(rows, a_mat, preferred_element_type=_F32)
          + jnp.dot(rsh * rmask, b_mat, preferred_element_type=_F32)
          + b1_ref[...])
    h = jnp.transpose(jax.nn.relu(y1)).reshape(64, B, 128)

    r, c = _iotas((256, 256))
    w2 = _dot(w2_ref[...], ((c == (r % 4) * 64 + r // 4)).astype(_F32))
    zp = jnp.zeros((64, B, 1), _F32)
    xp = jnp.concatenate([zp, h, zp], axis=2)
    xf = xp.reshape(64 * B, 130)
    phases = [jnp.dot(xf, _sel(130, 64, 2, d),
                      preferred_element_type=_F32).reshape(64, B, 64)
              for d in range(4)]
    xi = jnp.concatenate(phases, axis=0).reshape(256, B * 64)
    h = jax.nn.relu(_dot(w2, xi) + b2_ref[...])
    h = h.reshape(128, B, 64)

    h = _conv3(h, _taps3(w3_ref[...]), b3_ref[...])
    h = _res_block(h, er1a_ref[...], er1b_ref[...])
    h = _res_block(h, er2a_ref[...], er2b_ref[...])
    h = jax.nn.relu(h)

    z = _dot(wp_ref[...], h.reshape(128, B * 64)) + bp_ref[...]

    e = e_ref[...]
    esq = jnp.sum(e * e, axis=1, keepdims=True)
    zsq = jnp.sum(z * z, axis=0, keepdims=True)
    s = esq + zsq - 2.0 * _dot(e, z)
    idx = jnp.argmin(s, axis=0)
    iota = jax.lax.broadcasted_iota(jnp.int32, s.shape, 0)
    hot = (iota == idx[None, :]).astype(_F32)
    q = _dotT(e, hot)

    sq = jnp.sum((q - z) ** 2)
    hist = jnp.sum(hot, axis=1, keepdims=True)

    @pl.when(pid == 0)
    def _init():
        acc_sq[...] = jnp.zeros((1, 1), _F32)
        acc_hist[...] = jnp.zeros((128, 1), _F32)

    acc_sq[...] += sq.reshape(1, 1)
    acc_hist[...] += hist

    h = q.reshape(128, B, 64)
    h = _conv3(h, _taps3(dc1_ref[...]), dc1b_ref[...])
    h = _res_block(h, dr1a_ref[...], dr1b_ref[...])
    h = _res_block(h, dr2a_ref[...], dr2b_ref[...])
    h = jax.nn.relu(h)

    r, c = _iotas((256, 64))
    wt1 = wt1_ref[...]
    vq = [_dot(wt1, ((r == 4 * c + d)).astype(_F32)) for d in range(4)]
    xf = h.reshape(128, B * 64)
    p0 = _dotT(vq[0], xf).reshape(64, B, 64)
    p1 = _dotT(vq[1], xf).reshape(64, B, 64)
    p2 = _dotT(vq[2], xf).reshape(64, B, 64)
    p3 = _dotT(vq[3], xf).reshape(64, B, 64)
    zc = jnp.zeros((64, B, 1), _F32)
    p3s = jnp.concatenate([zc, p3[:, :, :63]], axis=2)
    p0s = jnp.concatenate([p0[:, :, 1:], zc], axis=2)
    y_even = (p1 + p3s).reshape(64 * B, 64)
    y_odd = (p0s + p2).reshape(64 * B, 64)
    y = (jnp.dot(y_even, _sel(128, 64, 2, 0).T, preferred_element_type=_F32)
         + jnp.dot(y_odd, _sel(128, 64, 2, 1).T, preferred_element_type=_F32))
    h = jax.nn.relu(y.reshape(64, B * 128) + t1b_ref[...]).reshape(64, B, 128)

    r, c = _iotas((240, 60))
    wt2 = wt2_ref[...]
    xf = h.reshape(64, B * 128)
    y = jnp.zeros((60 * B, 512), _F32)
    for d in range(4):
        uq = _dot(wt2, ((r == 4 * c + d)).astype(_F32))
        pd = _dotT(uq, xf).reshape(60 * B, 128)
        y = y + jnp.dot(pd, _sel(512, 128, 4, d).T,
                        preferred_element_type=_F32)
    h = (y.reshape(60, B * 512) + t2b_ref[...]).reshape(60, B, 512)

    taps = _taps3(wadj_ref[...])
    xf = h.reshape(60, B * 512)
    out = None
    for d in range(3):
        zd = _dot(taps[d], xf).reshape(60, B, 512)
        zl = jnp.zeros((60, B, 3 - d), _F32)
        zr = jnp.zeros((60, B, 1 + d), _F32)
        term = jnp.concatenate([zl, zd, zr], axis=2)
        out = term if out is None else out + term
    out = out.reshape(60, B * 516) + badj_ref[...]
    xout_ref[...] = jnp.transpose(out).reshape(B, 516, 60)

    @pl.when(pid == _GRID - 1)
    def _finalize():
        total = 128.0 * 64.0 * 128.0
        loss_ref[...] = 2.0 * acc_sq[...] / total
        probs = acc_hist[...] / 8192.0
        ent = jnp.sum(probs * jnp.log(probs + 1e-10))
        perp_ref[...] = jnp.exp(-ent).reshape(1, 1)


def _full(shape):
    nd = len(shape)
    return pl.BlockSpec(shape, lambda i: (0,) * nd)


def kernel(x, params):
    p = params
    n = x.shape[0]
    xcol = x.reshape(n, 128, 240)

    er = p['enc_res']
    dr = p['dec_res']
    ins = [xcol,
           p['enc_c1_w'].reshape(64, 240), p['enc_c1_b'].reshape(1, 64),
           p['enc_c2_w'].reshape(128, 256), p['enc_c2_b'].reshape(128, 1),
           p['enc_c3_w'].reshape(128, 384), p['enc_c3_b'].reshape(128, 1),
           er[0][0].reshape(32, 384), er[0][1].reshape(128, 32),
           er[1][0].reshape(32, 384), er[1][1].reshape(128, 32),
           p['pre_w'].reshape(128, 128), p['pre_b'].reshape(128, 1),
           p['codebook'],
           p['dec_c1_w'].reshape(128, 384), p['dec_c1_b'].reshape(128, 1),
           dr[0][0].reshape(32, 384), dr[0][1].reshape(128, 32),
           dr[1][0].reshape(32, 384), dr[1][1].reshape(128, 32),
           p['dec_t1_w'].reshape(128, 256), p['dec_t1_b'].reshape(64, 1),
           p['dec_t2_w'].reshape(64, 240), p['dec_t2_b'].reshape(60, 1),
           p['dec_adj_w'].reshape(60, 180), p['dec_adj_b'].reshape(60, 1)]

    in_specs = [pl.BlockSpec((_B, 128, 240), lambda i: (i, 0, 0))]
    in_specs += [_full(a.shape) for a in ins[1:]]

    out_shape = [jax.ShapeDtypeStruct((1, 1), _F32),
                 jax.ShapeDtypeStruct((128, 516, 60), _F32),
                 jax.ShapeDtypeStruct((1, 1), _F32)]
    out_specs = [pl.BlockSpec((1, 1), lambda i: (0, 0)),
                 pl.BlockSpec((_B, 516, 60), lambda i: (i, 0, 0)),
                 pl.BlockSpec((1, 1), lambda i: (0, 0))]

    loss, xo, perp = pl.pallas_call(
        _fwd,
        grid=(_GRID,),
        in_specs=in_specs,
        out_specs=out_specs,
        out_shape=out_shape,
        scratch_shapes=[pltpu.VMEM((1, 1), _F32),
                        pltpu.VMEM((128, 1), _F32)],
    )(*ins)

    return loss[0, 0], xo, perp[0, 0]


# speedup vs baseline: 3.1122x; 1.0076x over previous
import jax
import jax.numpy as jnp
from jax.experimental import pallas as pl
from jax.experimental.pallas import tpu as pltpu

_B = 16
_GRID = 8
_F32 = jnp.float32


def _dot(a, b):
    return jax.lax.dot_general(a, b, (((1,), (0,)), ((), ())),
                               preferred_element_type=_F32)


def _dotT(a, b):
    return jax.lax.dot_general(a, b, (((0,), (0,)), ((), ())),
                               preferred_element_type=_F32)


def _dotBT(a, b):
    return jax.lax.dot_general(a, b, (((1,), (1,)), ((), ())),
                               preferred_element_type=_F32)


def _iotas(shape):
    return (jax.lax.broadcasted_iota(jnp.int32, shape, 0),
            jax.lax.broadcasted_iota(jnp.int32, shape, 1))


def _sel(rows, cols, stride, offset):
    r, c = _iotas((rows, cols))
    return (r == stride * c + offset).astype(_F32)


def _taps3(flat):
    o, c3 = flat.shape
    c = c3 // 3
    r, cc = _iotas((c3, c))
    return [_dot(flat, ((r == 3 * cc + d)).astype(_F32)) for d in range(3)]


def _shift64(z2d, delta):
    o, bt = z2d.shape
    zf = z2d.reshape(o * (bt // 128), 128)
    r, c = _iotas((128, 128))
    m = ((r == c - delta) & (r // 64 == c // 64)).astype(_F32)
    return jnp.dot(zf, m, preferred_element_type=_F32).reshape(o, bt)


def _conv3(x, taps, bias=None):
    c, b, t = x.shape
    xf = x.reshape(c, b * t)
    z0 = _dot(taps[0], xf)
    z1 = _dot(taps[1], xf)
    z2 = _dot(taps[2], xf)
    o = z0.shape[0]
    y = (z1 + _shift64(z0, 1) + _shift64(z2, -1)).reshape(o, b, t)
    if bias is not None:
        y = y + bias[:, None, :1]
    return y


def _res_block(x, wa_flat, wb):
    c, b, t = x.shape
    h = jax.nn.relu(x)
    h = jax.nn.relu(_conv3(h, _taps3(wa_flat)))
    h = _dot(wb, h.reshape(-1, b * t))
    return x + h.reshape(c, b, t)


def _fwd(x_ref, w1_ref, b1_ref, w2_ref, b2_ref, w3_ref, b3_ref,
         er1a_ref, er1b_ref, er2a_ref, er2b_ref, wp_ref, bp_ref, e_ref,
         dc1_ref, dc1b_ref, dr1a_ref, dr1b_ref, dr2a_ref, dr2b_ref,
         wt1_ref, t1b_ref, wt2_ref, t2b_ref, wadj_ref, badj_ref,
         loss_ref, xout_ref, perp_ref, acc_sq, acc_hist):
    pid = pl.program_id(0)
    B = _B

    rows = x_ref[...].reshape(B * 512, 60)
    w1t = jnp.transpose(w1_ref[...])
    r, c = _iotas((60, 240))
    taps1 = [jnp.dot((c == 4 * r + d).astype(_F32), w1t,
                     preferred_element_type=_F32) for d in range(4)]
    z = [jnp.dot(rows, taps1[d], preferred_element_type=_F32)
         for d in range(4)]
    nrows = B * 512
    z0s = jnp.concatenate([jnp.zeros((1, 64), _F32), z[0][:nrows - 1]],
                          axis=0)
    rmask = (jax.lax.broadcasted_iota(jnp.int32, (nrows, 1), 0) % 512
             != 0).astype(_F32)
    u = (z0s * rmask + z[1]
         + jnp.concatenate([z[2][1:], jnp.zeros((1, 64), _F32)], axis=0)
         + jnp.concatenate([z[3][2:], jnp.zeros((2, 64), _F32)], axis=0))
    ut = jnp.transpose(u).reshape(64 * B, 512)
    h = jnp.dot(ut, _sel(512, 128, 4, 0), preferred_element_type=_F32)
    h = jax.nn.relu(h.reshape(64, B, 128) + b1_ref[...][:, None, :1])

    r, c = _iotas((256, 256))
    w2 = _dot(w2_ref[...], ((c == (r % 4) * 64 + r // 4)).astype(_F32))
    hf = h.reshape(64 * B, 128)
    phases = [jnp.dot(hf, _sel(128, 64, 2, d - 1),
                      preferred_element_type=_F32).reshape(64, B, 64)
              for d in range(4)]
    xi = jnp.concatenate(phases, axis=0).reshape(256, B * 64)
    h = jax.nn.relu(_dot(w2, xi) + b2_ref[...])
    h = h.reshape(128, B, 64)

    h = _conv3(h, _taps3(w3_ref[...]), b3_ref[...])
    h = _res_block(h, er1a_ref[...], er1b_ref[...])
    h = _res_block(h, er2a_ref[...], er2b_ref[...])
    h = jax.nn.relu(h)

    z = _dot(wp_ref[...], h.reshape(128, B * 64)) + bp_ref[...]

    e = e_ref[...]
    esq = jnp.sum(e * e, axis=1, keepdims=True)
    zsq = jnp.sum(z * z, axis=0, keepdims=True)
    s = esq + zsq - 2.0 * _dot(e, z)
    idx = jnp.argmin(s, axis=0)
    iota = jax.lax.broadcasted_iota(jnp.int32, s.shape, 0)
    hot = (iota == idx[None, :]).astype(_F32)
    q = _dotT(e, hot)

    sq = jnp.sum((q - z) ** 2)
    hist = jnp.sum(hot, axis=1, keepdims=True)

    @pl.when(pid == 0)
    def _init():
        acc_sq[...] = jnp.zeros((1, 1), _F32)
        acc_hist[...] = jnp.zeros((128, 1), _F32)

    acc_sq[...] += sq.reshape(1, 1)
    acc_hist[...] += hist

    h = q.reshape(128, B, 64)
    h = _conv3(h, _taps3(dc1_ref[...]), dc1b_ref[...])
    h = _res_block(h, dr1a_ref[...], dr1b_ref[...])
    h = _res_block(h, dr2a_ref[...], dr2b_ref[...])
    h = jax.nn.relu(h)

    r, c = _iotas((256, 64))
    wt1 = wt1_ref[...]
    vq = [_dot(wt1, ((r == 4 * c + d)).astype(_F32)) for d in range(4)]
    xf = h.reshape(128, B * 64)
    r, c = _iotas((128, 256))
    def _place(off):
        return ((c == (r // 64) * 128 + 2 * (r % 64) + off)
                & (c // 128 == r // 64)).astype(_F32)
    pr = [_dotT(v, xf).reshape(64 * B // 2, 128) for v in vq]
    y = (jnp.dot(pr[1], _place(0), preferred_element_type=_F32)
         + jnp.dot(pr[3], _place(2), preferred_element_type=_F32)
         + jnp.dot(pr[0], _place(-1), preferred_element_type=_F32)
         + jnp.dot(pr[2], _place(1), preferred_element_type=_F32))
    h = jax.nn.relu(y.reshape(64, B * 128) + t1b_ref[...]).reshape(64, B, 128)

    r, c = _iotas((240, 60))
    wt2 = wt2_ref[...]
    xf = h.reshape(64, B * 128)
    y = jnp.zeros((60 * B, 512), _F32)
    for d in range(4):
        uq = _dot(wt2, ((r == 4 * c + d)).astype(_F32))
        pd = _dotT(uq, xf).reshape(60 * B, 128)
        y = y + jnp.dot(pd, _sel(512, 128, 4, d).T,
                        preferred_element_type=_F32)
    h = (y.reshape(60, B * 512) + t2b_ref[...])

    hr = jnp.transpose(h)
    taps = _taps3(wadj_ref[...])
    out = None
    for d in range(3):
        yd = _dotBT(hr, taps[d]).reshape(B, 512, 60)
        term = jnp.concatenate([jnp.zeros((B, 3 - d, 60), _F32), yd,
                                jnp.zeros((B, 1 + d, 60), _F32)], axis=1)
        out = term if out is None else out + term
    xout_ref[...] = out + badj_ref[...]

    @pl.when(pid == _GRID - 1)
    def _finalize():
        total = 128.0 * 64.0 * 128.0
        loss_ref[...] = 2.0 * acc_sq[...] / total
        probs = acc_hist[...] / 8192.0
        ent = jnp.sum(probs * jnp.log(probs + 1e-10))
        perp_ref[...] = jnp.exp(-ent).reshape(1, 1)


def _full(shape):
    nd = len(shape)
    return pl.BlockSpec(shape, lambda i: (0,) * nd)


def kernel(x, params):
    p = params
    er = p['enc_res']
    dr = p['dec_res']
    ins = [x,
           p['enc_c1_w'].reshape(64, 240), p['enc_c1_b'].reshape(64, 1),
           p['enc_c2_w'].reshape(128, 256), p['enc_c2_b'].reshape(128, 1),
           p['enc_c3_w'].reshape(128, 384), p['enc_c3_b'].reshape(128, 1),
           er[0][0].reshape(32, 384), er[0][1].reshape(128, 32),
           er[1][0].reshape(32, 384), er[1][1].reshape(128, 32),
           p['pre_w'].reshape(128, 128), p['pre_b'].reshape(128, 1),
           p['codebook'],
           p['dec_c1_w'].reshape(128, 384), p['dec_c1_b'].reshape(128, 1),
           dr[0][0].reshape(32, 384), dr[0][1].reshape(128, 32),
           dr[1][0].reshape(32, 384), dr[1][1].reshape(128, 32),
           p['dec_t1_w'].reshape(128, 256), p['dec_t1_b'].reshape(64, 1),
           p['dec_t2_w'].reshape(64, 240), p['dec_t2_b'].reshape(60, 1),
           p['dec_adj_w'].reshape(60, 180), p['dec_adj_b'].reshape(1, 1, 60)]

    in_specs = [pl.BlockSpec((_B, 512, 60), lambda i: (i, 0, 0))]
    in_specs += [_full(a.shape) for a in ins[1:]]

    out_shape = [jax.ShapeDtypeStruct((1, 1), _F32),
                 jax.ShapeDtypeStruct((128, 516, 60), _F32),
                 jax.ShapeDtypeStruct((1, 1), _F32)]
    out_specs = [pl.BlockSpec((1, 1), lambda i: (0, 0)),
                 pl.BlockSpec((_B, 516, 60), lambda i: (i, 0, 0)),
                 pl.BlockSpec((1, 1), lambda i: (0, 0))]

    loss, xo, perp = pl.pallas_call(
        _fwd,
        grid=(_GRID,),
        in_specs=in_specs,
        out_specs=out_specs,
        out_shape=out_shape,
        scratch_shapes=[pltpu.VMEM((1, 1), _F32),
                        pltpu.VMEM((128, 1), _F32)],
    )(*ins)

    return loss[0, 0], xo, perp[0, 0]
